# initial kernel scaffold (unmeasured)
import jax
import jax.numpy as jnp
from jax import lax
from jax.experimental import pallas as pl
from jax.experimental.pallas import tpu as pltpu


def kernel(
    x,
):
    def body(*refs):
        pass

    out_shape = jax.ShapeDtypeStruct(..., jnp.float32)
    return pl.pallas_call(body, out_shape=out_shape)(...)


# baseline (device time: 17592 ns/iter reference)
import jax
import jax.numpy as jnp
from jax import lax
from jax.experimental import pallas as pl
from jax.experimental.pallas import tpu as pltpu


def kernel(x):
    m, n = x.shape

    def body(x_ref, out_ref, recv_ref, send_sem, recv_sem):
        my_x = lax.axis_index("x")
        my_y = lax.axis_index("y")
        my_z = lax.axis_index("z")
        partner = (1 - my_x, my_y, my_z)

        barrier_sem = pltpu.get_barrier_semaphore()
        pl.semaphore_signal(
            barrier_sem, inc=1,
            device_id=partner, device_id_type=pl.DeviceIdType.MESH,
        )
        pl.semaphore_wait(barrier_sem, 1)

        rdma = pltpu.make_async_remote_copy(
            src_ref=x_ref,
            dst_ref=recv_ref,
            send_sem=send_sem,
            recv_sem=recv_sem,
            device_id=partner,
            device_id_type=pl.DeviceIdType.MESH,
        )
        rdma.start()
        rdma.wait()

        out_ref[...] = x_ref[...] + recv_ref[...]

    return pl.pallas_call(
        body,
        out_shape=jax.ShapeDtypeStruct((m, n), x.dtype),
        in_specs=[pl.BlockSpec(memory_space=pltpu.VMEM)],
        out_specs=pl.BlockSpec(memory_space=pltpu.VMEM),
        scratch_shapes=[
            pltpu.VMEM((m, n), x.dtype),
            pltpu.SemaphoreType.DMA,
            pltpu.SemaphoreType.DMA,
        ],
        compiler_params=pltpu.CompilerParams(collective_id=0),
    )(x)


# device time: 15349 ns/iter; 1.1461x vs baseline; 1.1461x over previous
import jax
import jax.numpy as jnp
from jax import lax
from jax.experimental import pallas as pl
from jax.experimental.pallas import tpu as pltpu


def kernel(x):
    m, n = x.shape
    qm = m // 4

    def body(
        x_ref,
        out_ref,
        send_a,
        send_b,
        recv_a,
        recv_b,
        sum_a,
        recv_y,
        recv_z,
        send_sems,
        recv_sems,
    ):
        my_x = lax.axis_index("x")
        my_y = lax.axis_index("y")
        my_z = lax.axis_index("z")
        x_partner = (1 - my_x, my_y, my_z)
        y_partner = (my_x, 1 - my_y, my_z)
        z_partner = (my_x, my_y, 1 - my_z)

        q_own = 2 * my_y + my_z
        q_diag = 3 - q_own
        q_from_y = 2 * (1 - my_y) + my_z
        q_from_z = 2 * my_y + (1 - my_z)

        barrier_sem = pltpu.get_barrier_semaphore()
        for nbr in (x_partner, y_partner, z_partner):
            pl.semaphore_signal(
                barrier_sem, inc=1,
                device_id=nbr, device_id_type=pl.DeviceIdType.MESH,
            )
        pl.semaphore_wait(barrier_sem, 3)

        send_a[...] = x_ref[pl.ds(q_own * qm, qm), :]
        send_b[...] = x_ref[pl.ds(q_diag * qm, qm), :]

        rdma_xa = pltpu.make_async_remote_copy(
            src_ref=send_a, dst_ref=recv_a,
            send_sem=send_sems.at[0], recv_sem=recv_sems.at[0],
            device_id=x_partner, device_id_type=pl.DeviceIdType.MESH,
        )
        rdma_xb = pltpu.make_async_remote_copy(
            src_ref=send_b, dst_ref=recv_b,
            send_sem=send_sems.at[1], recv_sem=recv_sems.at[1],
            device_id=x_partner, device_id_type=pl.DeviceIdType.MESH,
        )
        rdma_xa.start()
        rdma_xb.start()

        rdma_xa.wait_recv()
        sum_a[...] = send_a[...] + recv_a[...]
        out_ref[pl.ds(q_own * qm, qm), :] = sum_a[...]

        rdma_y = pltpu.make_async_remote_copy(
            src_ref=sum_a, dst_ref=recv_y,
            send_sem=send_sems.at[2], recv_sem=recv_sems.at[2],
            device_id=y_partner, device_id_type=pl.DeviceIdType.MESH,
        )
        rdma_z = pltpu.make_async_remote_copy(
            src_ref=sum_a, dst_ref=recv_z,
            send_sem=send_sems.at[3], recv_sem=recv_sems.at[3],
            device_id=z_partner, device_id_type=pl.DeviceIdType.MESH,
        )
        rdma_y.start()
        rdma_z.start()

        rdma_xb.wait_recv()
        out_ref[pl.ds(q_diag * qm, qm), :] = send_b[...] + recv_b[...]

        rdma_y.wait_recv()
        out_ref[pl.ds(q_from_y * qm, qm), :] = recv_y[...]
        rdma_z.wait_recv()
        out_ref[pl.ds(q_from_z * qm, qm), :] = recv_z[...]

        rdma_xa.wait_send()
        rdma_xb.wait_send()
        rdma_y.wait_send()
        rdma_z.wait_send()

    return pl.pallas_call(
        body,
        out_shape=jax.ShapeDtypeStruct((m, n), x.dtype),
        in_specs=[pl.BlockSpec(memory_space=pltpu.VMEM)],
        out_specs=pl.BlockSpec(memory_space=pltpu.VMEM),
        scratch_shapes=[
            pltpu.VMEM((qm, n), x.dtype),
            pltpu.VMEM((qm, n), x.dtype),
            pltpu.VMEM((qm, n), x.dtype),
            pltpu.VMEM((qm, n), x.dtype),
            pltpu.VMEM((qm, n), x.dtype),
            pltpu.VMEM((qm, n), x.dtype),
            pltpu.VMEM((qm, n), x.dtype),
            pltpu.SemaphoreType.DMA((4,)),
            pltpu.SemaphoreType.DMA((4,)),
        ],
        compiler_params=pltpu.CompilerParams(collective_id=0),
    )(x)


# device time: 13964 ns/iter; 1.2598x vs baseline; 1.0992x over previous
import jax
import jax.numpy as jnp
from jax import lax
from jax.experimental import pallas as pl
from jax.experimental.pallas import tpu as pltpu

N_CHUNK = 2


def kernel(x):
    m, n = x.shape
    qm = m // 4
    cm = qm // N_CHUNK

    def body(
        x_ref,
        out_ref,
        recv_a,
        recv_b,
        sum_a,
        recv_y,
        recv_z,
        send_sems,
        recv_sems,
    ):
        my_x = lax.axis_index("x")
        my_y = lax.axis_index("y")
        my_z = lax.axis_index("z")
        x_partner = (1 - my_x, my_y, my_z)
        y_partner = (my_x, 1 - my_y, my_z)
        z_partner = (my_x, my_y, 1 - my_z)

        q_own = 2 * my_y + my_z
        q_diag = 3 - q_own
        q_from_y = 2 * (1 - my_y) + my_z
        q_from_z = 2 * my_y + (1 - my_z)

        barrier_sem = pltpu.get_barrier_semaphore()
        for nbr in (x_partner, y_partner, z_partner):
            pl.semaphore_signal(
                barrier_sem, inc=1,
                device_id=nbr, device_id_type=pl.DeviceIdType.MESH,
            )
        pl.semaphore_wait(barrier_sem, 3)

        rdma_xa = []
        for c in range(N_CHUNK):
            r = pltpu.make_async_remote_copy(
                src_ref=x_ref.at[pl.ds(q_own * qm + c * cm, cm), :],
                dst_ref=recv_a.at[pl.ds(c * cm, cm), :],
                send_sem=send_sems.at[c],
                recv_sem=recv_sems.at[c],
                device_id=x_partner,
                device_id_type=pl.DeviceIdType.MESH,
            )
            r.start()
            rdma_xa.append(r)
        rdma_xb = pltpu.make_async_remote_copy(
            src_ref=x_ref.at[pl.ds(q_diag * qm, qm), :],
            dst_ref=recv_b,
            send_sem=send_sems.at[N_CHUNK],
            recv_sem=recv_sems.at[N_CHUNK],
            device_id=x_partner,
            device_id_type=pl.DeviceIdType.MESH,
        )
        rdma_xb.start()

        rdma_yz = []
        for c in range(N_CHUNK):
            rdma_xa[c].wait_recv()
            sum_a[pl.ds(c * cm, cm), :] = (
                x_ref[pl.ds(q_own * qm + c * cm, cm), :]
                + recv_a[pl.ds(c * cm, cm), :]
            )
            ry = pltpu.make_async_remote_copy(
                src_ref=sum_a.at[pl.ds(c * cm, cm), :],
                dst_ref=recv_y.at[pl.ds(c * cm, cm), :],
                send_sem=send_sems.at[N_CHUNK + 1 + 2 * c],
                recv_sem=recv_sems.at[N_CHUNK + 1 + 2 * c],
                device_id=y_partner,
                device_id_type=pl.DeviceIdType.MESH,
            )
            rz = pltpu.make_async_remote_copy(
                src_ref=sum_a.at[pl.ds(c * cm, cm), :],
                dst_ref=recv_z.at[pl.ds(c * cm, cm), :],
                send_sem=send_sems.at[N_CHUNK + 2 + 2 * c],
                recv_sem=recv_sems.at[N_CHUNK + 2 + 2 * c],
                device_id=z_partner,
                device_id_type=pl.DeviceIdType.MESH,
            )
            ry.start()
            rz.start()
            rdma_yz.append((ry, rz))

        out_ref[pl.ds(q_own * qm, qm), :] = sum_a[...]

        rdma_xb.wait_recv()
        out_ref[pl.ds(q_diag * qm, qm), :] = (
            x_ref[pl.ds(q_diag * qm, qm), :] + recv_b[...]
        )

        for ry, rz in rdma_yz:
            ry.wait_recv()
            rz.wait_recv()
        out_ref[pl.ds(q_from_y * qm, qm), :] = recv_y[...]
        out_ref[pl.ds(q_from_z * qm, qm), :] = recv_z[...]

        for r in rdma_xa:
            r.wait_send()
        rdma_xb.wait_send()
        for ry, rz in rdma_yz:
            ry.wait_send()
            rz.wait_send()

    n_sems = 1 + N_CHUNK + 2 * N_CHUNK
    return pl.pallas_call(
        body,
        out_shape=jax.ShapeDtypeStruct((m, n), x.dtype),
        in_specs=[pl.BlockSpec(memory_space=pltpu.VMEM)],
        out_specs=pl.BlockSpec(memory_space=pltpu.VMEM),
        scratch_shapes=[
            pltpu.VMEM((qm, n), x.dtype),
            pltpu.VMEM((qm, n), x.dtype),
            pltpu.VMEM((qm, n), x.dtype),
            pltpu.VMEM((qm, n), x.dtype),
            pltpu.VMEM((qm, n), x.dtype),
            pltpu.SemaphoreType.DMA((n_sems,)),
            pltpu.SemaphoreType.DMA((n_sems,)),
        ],
        compiler_params=pltpu.CompilerParams(collective_id=0),
    )(x)


# device time: 13941 ns/iter; 1.2619x vs baseline; 1.0016x over previous
import jax
import jax.numpy as jnp
from jax import lax
from jax.experimental import pallas as pl
from jax.experimental.pallas import tpu as pltpu

N_CHUNK = 2


def kernel(x):
    m, n = x.shape
    qm = m // 4
    cm = qm // N_CHUNK

    def body(
        x_ref,
        out_ref,
        recv_a,
        recv_b,
        send_sems,
        recv_sems,
    ):
        my_x = lax.axis_index("x")
        my_y = lax.axis_index("y")
        my_z = lax.axis_index("z")
        x_partner = (1 - my_x, my_y, my_z)
        y_partner = (my_x, 1 - my_y, my_z)
        z_partner = (my_x, my_y, 1 - my_z)

        q_own = 2 * my_y + my_z
        q_diag = 3 - q_own

        barrier_sem = pltpu.get_barrier_semaphore()
        for nbr in (x_partner, y_partner, z_partner):
            pl.semaphore_signal(
                barrier_sem, inc=1,
                device_id=nbr, device_id_type=pl.DeviceIdType.MESH,
            )
        pl.semaphore_wait(barrier_sem, 3)

        rdma_xa = []
        for c in range(N_CHUNK):
            r = pltpu.make_async_remote_copy(
                src_ref=x_ref.at[pl.ds(q_own * qm + c * cm, cm), :],
                dst_ref=recv_a.at[pl.ds(c * cm, cm), :],
                send_sem=send_sems.at[c],
                recv_sem=recv_sems.at[c],
                device_id=x_partner,
                device_id_type=pl.DeviceIdType.MESH,
            )
            r.start()
            rdma_xa.append(r)
        rdma_xb = pltpu.make_async_remote_copy(
            src_ref=x_ref.at[pl.ds(q_diag * qm, qm), :],
            dst_ref=recv_b,
            send_sem=send_sems.at[N_CHUNK],
            recv_sem=recv_sems.at[N_CHUNK],
            device_id=x_partner,
            device_id_type=pl.DeviceIdType.MESH,
        )
        rdma_xb.start()

        rdma_yz = []
        for c in range(N_CHUNK):
            rows = pl.ds(q_own * qm + c * cm, cm)
            rdma_xa[c].wait_recv()
            out_ref[rows, :] = x_ref[rows, :] + recv_a[pl.ds(c * cm, cm), :]
            ry = pltpu.make_async_remote_copy(
                src_ref=out_ref.at[rows, :],
                dst_ref=out_ref.at[rows, :],
                send_sem=send_sems.at[N_CHUNK + 1 + 2 * c],
                recv_sem=recv_sems.at[N_CHUNK + 1 + 2 * c],
                device_id=y_partner,
                device_id_type=pl.DeviceIdType.MESH,
            )
            rz = pltpu.make_async_remote_copy(
                src_ref=out_ref.at[rows, :],
                dst_ref=out_ref.at[rows, :],
                send_sem=send_sems.at[N_CHUNK + 2 + 2 * c],
                recv_sem=recv_sems.at[N_CHUNK + 2 + 2 * c],
                device_id=z_partner,
                device_id_type=pl.DeviceIdType.MESH,
            )
            ry.start()
            rz.start()
            rdma_yz.append((ry, rz))

        rdma_xb.wait_recv()
        out_ref[pl.ds(q_diag * qm, qm), :] = (
            x_ref[pl.ds(q_diag * qm, qm), :] + recv_b[...]
        )

        for ry, rz in rdma_yz:
            ry.wait_recv()
            rz.wait_recv()

        for r in rdma_xa:
            r.wait_send()
        rdma_xb.wait_send()
        for ry, rz in rdma_yz:
            ry.wait_send()
            rz.wait_send()

    n_sems = 1 + N_CHUNK + 2 * N_CHUNK
    return pl.pallas_call(
        body,
        out_shape=jax.ShapeDtypeStruct((m, n), x.dtype),
        in_specs=[pl.BlockSpec(memory_space=pltpu.VMEM)],
        out_specs=pl.BlockSpec(memory_space=pltpu.VMEM),
        scratch_shapes=[
            pltpu.VMEM((qm, n), x.dtype),
            pltpu.VMEM((qm, n), x.dtype),
            pltpu.SemaphoreType.DMA((n_sems,)),
            pltpu.SemaphoreType.DMA((n_sems,)),
        ],
        compiler_params=pltpu.CompilerParams(collective_id=0),
    )(x)
